# baseline (speedup 1.0000x reference)
import functools

import jax
import jax.numpy as jnp
from jax import lax
from jax.experimental import pallas as pl
from jax.experimental.pallas import tpu as pltpu
from jax.experimental.pallas import tpu_sc as plsc

VOCAB_N = 1000000
EMBED = 64
ROW = 128
B_TOT = 4096 * 200
NW = 32
B_PER_W = B_TOT // NW
CHUNK = 512
N_CHUNKS = B_PER_W // CHUNK
NBUF = 2
N_ROUNDS = N_CHUNKS // NBUF

NBLK = VOCAB_N // 128
TAIL_C0 = VOCAB_N - 128

_mesh = plsc.VectorSubcoreMesh(core_axis_name="c", subcore_axis_name="s")


@functools.partial(
    pl.kernel,
    mesh=_mesh,
    out_type=jax.ShapeDtypeStruct((VOCAB_N * EMBED,), jnp.float32),
    scratch_types=[pltpu.VMEM((EMBED, 128), jnp.float32)] * 2
    + [pltpu.VMEM((128 * EMBED,), jnp.float32)] * 2
    + [pltpu.SemaphoreType.DMA] * 4,
    compiler_params=pltpu.CompilerParams(
        use_tc_tiling_on_sc=True, needs_layout_passes=False
    ),
)
def _transpose_table(tt_hbm, tail_hbm, out_hbm, s0, s1, d0, d1, *sems):
    svmem = (s0, s1)
    dvmem = (d0, d1)
    i_sem = sems[0:2]
    o_sem = sems[2:4]

    wid = lax.axis_index("s") * 2 + lax.axis_index("c")

    iota = lax.iota(jnp.int32, 16)
    jvs = [jg * 16 + iota for jg in range(4)]

    def c0_of(t):
        blk = jnp.minimum(wid + NW * t, NBLK - 1)
        return pl.multiple_of(blk * 128, 128)

    def in_copy(t, b):
        return pltpu.make_async_copy(
            tt_hbm.at[:, pl.ds(c0_of(t), 128)], svmem[b], i_sem[b]
        )

    def tail_in_copy(b):
        return pltpu.make_async_copy(tail_hbm.at[:, :], svmem[b], i_sem[b])

    def out_copy(c0, b):
        return pltpu.make_async_copy(
            dvmem[b], out_hbm.at[pl.ds(c0 * EMBED, 128 * EMBED)], o_sem[b]
        )

    def transpose_block(b):
        s = svmem[b]
        d = dvmem[b]

        def tbody(i16, carry):
            for il in range(16):
                i = i16 * 16 + il
                iv = jnp.full((16,), 0, jnp.int32) + i
                for jg in range(4):
                    v = plsc.load_gather(s, [jvs[jg], iv])
                    d[pl.ds(i * EMBED + jg * 16, 16)] = v
            return carry

        lax.fori_loop(0, 8, tbody, 0)

    NT = -(-NBLK // NW)
    NPAIR = NT // 2

    in_copy(0, 0).start()

    def body(r, carry):
        for bb in range(2):
            t = r * 2 + bb
            in_copy(t, bb).wait()
            in_copy(t + 1, 1 - bb).start()

            @pl.when(t >= 2)
            def _():
                out_copy(c0_of(t - 2), bb).wait()

            transpose_block(bb)
            out_copy(c0_of(t), bb).start()
        return carry

    lax.fori_loop(0, NPAIR, body, 0)

    t_last = NT - 1

    @pl.when(wid == NW - 1)
    def _():
        tail_in_copy(1).start()

    in_copy(t_last, 0).wait()
    out_copy(c0_of(t_last - 2), 0).wait()
    transpose_block(0)
    out_copy(c0_of(t_last), 0).start()

    @pl.when(wid == NW - 1)
    def _():
        tail_in_copy(1).wait()
        out_copy(c0_of(t_last - 1), 1).wait()
        transpose_block(1)
        out_copy(TAIL_C0, 1).start()
        out_copy(TAIL_C0, 1).wait()

    @pl.when(wid != NW - 1)
    def _():
        out_copy(c0_of(t_last - 1), 1).wait()

    out_copy(c0_of(t_last), 0).wait()


@functools.partial(
    pl.kernel,
    mesh=_mesh,
    out_type=jax.ShapeDtypeStruct((B_TOT, ROW), jnp.float32),
    scratch_types=[pltpu.VMEM((CHUNK,), jnp.int32)] * NBUF
    + [pltpu.VMEM((CHUNK, EMBED), jnp.float32)] * NBUF
    + [pltpu.SemaphoreType.DMA] * (3 * NBUF),
    compiler_params=pltpu.CompilerParams(use_tc_tiling_on_sc=False),
)
def _embed_gather(table_hbm, idx_hbm, out_hbm, *scratch):
    idx_v = scratch[0:NBUF]
    rows_v = scratch[NBUF : 2 * NBUF]
    sems = scratch[2 * NBUF :]
    i_sem = sems[0:NBUF]
    g_sem = sems[NBUF : 2 * NBUF]
    s_sem = sems[2 * NBUF : 3 * NBUF]

    wid = lax.axis_index("s") * 2 + lax.axis_index("c")
    base = wid * B_PER_W

    def idx_copy(chunk, b):
        return pltpu.make_async_copy(
            idx_hbm.at[pl.ds(base + chunk * CHUNK, CHUNK)], idx_v[b], i_sem[b]
        )

    def gather_copy(b):
        return pltpu.make_async_copy(table_hbm.at[idx_v[b]], rows_v[b], g_sem[b])

    def store_copy(chunk, b):
        return pltpu.make_async_copy(
            rows_v[b],
            out_hbm.at[pl.ds(base + chunk * CHUNK, CHUNK), pl.ds(0, EMBED)],
            s_sem[b],
        )

    for b in range(NBUF):
        idx_copy(b, b).start()
    for b in range(NBUF):
        idx_copy(b, b).wait()
        gather_copy(b).start()

    def body(r, carry):
        for b in range(NBUF):
            g = r * NBUF + b
            gather_copy(b).wait()
            store_copy(g, b).start()
            idx_copy(g + NBUF, b).start()
            store_copy(g, b).wait()
            idx_copy(g + NBUF, b).wait()
            gather_copy(b).start()
        return carry

    lax.fori_loop(0, N_ROUNDS - 1, body, 0)

    last = (N_ROUNDS - 1) * NBUF
    for b in range(NBUF):
        gather_copy(b).wait()
        store_copy(last + b, b).start()
    for b in range(NBUF):
        store_copy(last + b, b).wait()


def kernel(sentence, table):
    idx = sentence.reshape(-1).astype(jnp.int32)
    tt = jnp.transpose(table)
    tail = lax.slice(tt, (0, TAIL_C0), (EMBED, VOCAB_N))
    flat = _transpose_table(tt, tail)
    t64 = flat.reshape(VOCAB_N, EMBED)
    out = _embed_gather(t64, idx)
    return out[:, :EMBED].reshape(sentence.shape + (EMBED,))


# speedup vs baseline: 1.6432x; 1.6432x over previous
import functools

import jax
import jax.numpy as jnp
from jax import lax
from jax.experimental import pallas as pl
from jax.experimental.pallas import tpu as pltpu
from jax.experimental.pallas import tpu_sc as plsc

VOCAB_N = 1000000
EMBED = 64
ROW = 128
B_TOT = 4096 * 200
NW = 32
B_PER_W = B_TOT // NW
CHUNK = 512
N_CHUNKS = B_PER_W // CHUNK
NBUF = 2
N_ROUNDS = N_CHUNKS // NBUF

NBLK = VOCAB_N // 128
TAIL_C0 = VOCAB_N - 128

_mesh = plsc.VectorSubcoreMesh(core_axis_name="c", subcore_axis_name="s")


@functools.partial(
    pl.kernel,
    mesh=_mesh,
    out_type=jax.ShapeDtypeStruct((VOCAB_N * EMBED,), jnp.float32),
    scratch_types=[pltpu.VMEM((EMBED, 128), jnp.float32)] * 2
    + [pltpu.VMEM((128 * EMBED,), jnp.float32)] * 2
    + [pltpu.SemaphoreType.DMA] * 4,
    compiler_params=pltpu.CompilerParams(
        use_tc_tiling_on_sc=True, needs_layout_passes=False
    ),
)
def _transpose_table(tt_hbm, tail_hbm, out_hbm, s0, s1, d0, d1, *sems):
    svmem = (s0, s1)
    dvmem = (d0, d1)
    i_sem = sems[0:2]
    o_sem = sems[2:4]

    wid = lax.axis_index("s") * 2 + lax.axis_index("c")

    iota = lax.iota(jnp.int32, 16)
    jvs = [jg * 16 + iota for jg in range(4)]

    def c0_of(t):
        blk = jnp.minimum(wid + NW * t, NBLK - 1)
        return pl.multiple_of(blk * 128, 128)

    def in_copy(t, b):
        return pltpu.make_async_copy(
            tt_hbm.at[:, pl.ds(c0_of(t), 128)], svmem[b], i_sem[b]
        )

    def tail_in_copy(b):
        return pltpu.make_async_copy(tail_hbm.at[:, :], svmem[b], i_sem[b])

    def out_copy(c0, b):
        return pltpu.make_async_copy(
            dvmem[b], out_hbm.at[pl.ds(c0 * EMBED, 128 * EMBED)], o_sem[b]
        )

    def transpose_block(b):
        s = svmem[b]
        d = dvmem[b]

        @plsc.parallel_loop(0, 128, step=1, unroll=8)
        def _(i):
            iv = jnp.full((16,), 0, jnp.int32) + i
            for jg in range(4):
                v = plsc.load_gather(s, [jvs[jg], iv])
                d[pl.ds(i * EMBED + jg * 16, 16)] = v

    NT = -(-NBLK // NW)
    NPAIR = NT // 2

    in_copy(0, 0).start()

    def body(r, carry):
        for bb in range(2):
            t = r * 2 + bb
            in_copy(t, bb).wait()
            in_copy(t + 1, 1 - bb).start()

            @pl.when(t >= 2)
            def _():
                out_copy(c0_of(t - 2), bb).wait()

            transpose_block(bb)
            out_copy(c0_of(t), bb).start()
        return carry

    lax.fori_loop(0, NPAIR, body, 0)

    t_last = NT - 1

    @pl.when(wid == NW - 1)
    def _():
        tail_in_copy(1).start()

    in_copy(t_last, 0).wait()
    out_copy(c0_of(t_last - 2), 0).wait()
    transpose_block(0)
    out_copy(c0_of(t_last), 0).start()

    @pl.when(wid == NW - 1)
    def _():
        tail_in_copy(1).wait()
        out_copy(c0_of(t_last - 1), 1).wait()
        transpose_block(1)
        out_copy(TAIL_C0, 1).start()
        out_copy(TAIL_C0, 1).wait()

    @pl.when(wid != NW - 1)
    def _():
        out_copy(c0_of(t_last - 1), 1).wait()

    out_copy(c0_of(t_last), 0).wait()


@functools.partial(
    pl.kernel,
    mesh=_mesh,
    out_type=jax.ShapeDtypeStruct((B_TOT, ROW), jnp.float32),
    scratch_types=[pltpu.VMEM((CHUNK,), jnp.int32)] * NBUF
    + [pltpu.VMEM((CHUNK, EMBED), jnp.float32)] * NBUF
    + [pltpu.SemaphoreType.DMA] * (3 * NBUF),
    compiler_params=pltpu.CompilerParams(use_tc_tiling_on_sc=False),
)
def _embed_gather(table_hbm, idx_hbm, out_hbm, *scratch):
    idx_v = scratch[0:NBUF]
    rows_v = scratch[NBUF : 2 * NBUF]
    sems = scratch[2 * NBUF :]
    i_sem = sems[0:NBUF]
    g_sem = sems[NBUF : 2 * NBUF]
    s_sem = sems[2 * NBUF : 3 * NBUF]

    wid = lax.axis_index("s") * 2 + lax.axis_index("c")
    base = wid * B_PER_W

    def idx_copy(chunk, b):
        return pltpu.make_async_copy(
            idx_hbm.at[pl.ds(base + chunk * CHUNK, CHUNK)], idx_v[b], i_sem[b]
        )

    def gather_copy(b):
        return pltpu.make_async_copy(table_hbm.at[idx_v[b]], rows_v[b], g_sem[b])

    def store_copy(chunk, b):
        return pltpu.make_async_copy(
            rows_v[b],
            out_hbm.at[pl.ds(base + chunk * CHUNK, CHUNK), pl.ds(0, EMBED)],
            s_sem[b],
        )

    for b in range(NBUF):
        idx_copy(b, b).start()
    for b in range(NBUF):
        idx_copy(b, b).wait()
        gather_copy(b).start()

    def body(r, carry):
        for b in range(NBUF):
            g = r * NBUF + b
            gather_copy(b).wait()
            store_copy(g, b).start()
            idx_copy(g + NBUF, b).start()
            store_copy(g, b).wait()
            idx_copy(g + NBUF, b).wait()
            gather_copy(b).start()
        return carry

    lax.fori_loop(0, N_ROUNDS - 1, body, 0)

    last = (N_ROUNDS - 1) * NBUF
    for b in range(NBUF):
        gather_copy(b).wait()
        store_copy(last + b, b).start()
    for b in range(NBUF):
        store_copy(last + b, b).wait()


def kernel(sentence, table):
    idx = sentence.reshape(-1).astype(jnp.int32)
    tt = jnp.transpose(table)
    tail = lax.slice(tt, (0, TAIL_C0), (EMBED, VOCAB_N))
    flat = _transpose_table(tt, tail)
    t64 = flat.reshape(VOCAB_N, EMBED)
    out = _embed_gather(t64, idx)
    return out[:, :EMBED].reshape(sentence.shape + (EMBED,))


# speedup vs baseline: 2.9054x; 1.7681x over previous
import functools

import jax
import jax.numpy as jnp
from jax import lax
from jax.experimental import pallas as pl
from jax.experimental.pallas import tpu as pltpu
from jax.experimental.pallas import tpu_sc as plsc

VOCAB_N = 1000000
EMBED = 64
ROW = 128
B_TOT = 4096 * 200
NW = 32
B_PER_W = B_TOT // NW
CHUNK = 512
N_CHUNKS = B_PER_W // CHUNK
NBUF = 2
N_ROUNDS = N_CHUNKS // NBUF

NBLK = VOCAB_N // 128
TAIL_C0 = VOCAB_N - 128

_mesh = plsc.VectorSubcoreMesh(core_axis_name="c", subcore_axis_name="s")


@functools.partial(
    pl.kernel,
    mesh=_mesh,
    out_type=jax.ShapeDtypeStruct((VOCAB_N * EMBED,), jnp.float32),
    scratch_types=[pltpu.VMEM((EMBED, 128), jnp.float32)] * 2
    + [pltpu.VMEM((128 * EMBED,), jnp.float32)] * 2
    + [pltpu.SemaphoreType.DMA] * 4,
    compiler_params=pltpu.CompilerParams(
        use_tc_tiling_on_sc=True, needs_layout_passes=False
    ),
)
def _transpose_table(tt_hbm, tail_hbm, out_hbm, s0, s1, d0, d1, *sems):
    svmem = (s0, s1)
    dvmem = (d0, d1)
    i_sem = sems[0:2]
    o_sem = sems[2:4]

    wid = lax.axis_index("s") * 2 + lax.axis_index("c")

    iota = lax.iota(jnp.int32, 16)
    jslots = [(iota + k) & 15 for k in range(16)]

    def c0_of(t):
        blk = jnp.minimum(wid + NW * t, NBLK - 1)
        return pl.multiple_of(blk * 128, 128)

    def in_copy(t, b):
        return pltpu.make_async_copy(
            tt_hbm.at[:, pl.ds(c0_of(t), 128)], svmem[b], i_sem[b]
        )

    def tail_in_copy(b):
        return pltpu.make_async_copy(tail_hbm.at[:, :], svmem[b], i_sem[b])

    def out_copy(c0, b):
        return pltpu.make_async_copy(
            dvmem[b], out_hbm.at[pl.ds(c0 * EMBED, 128 * EMBED)], o_sem[b]
        )

    def transpose_block(b):
        s = svmem[b]
        d = dvmem[b]

        @plsc.parallel_loop(0, 32, step=1, unroll=4)
        def _(sb):
            I = (sb % 8) * 16
            J = (sb // 8) * 16
            iv = I + iota
            dbase = iv * EMBED + J
            for k in range(16):
                jv = J + jslots[k]
                v = plsc.load_gather(s, [jv, iv])
                plsc.store_scatter(d, [dbase + jslots[k]], v)

    NT = -(-NBLK // NW)
    NPAIR = NT // 2

    in_copy(0, 0).start()

    def body(r, carry):
        for bb in range(2):
            t = r * 2 + bb
            in_copy(t, bb).wait()
            in_copy(t + 1, 1 - bb).start()

            @pl.when(t >= 2)
            def _():
                out_copy(c0_of(t - 2), bb).wait()

            transpose_block(bb)
            out_copy(c0_of(t), bb).start()
        return carry

    lax.fori_loop(0, NPAIR, body, 0)

    t_last = NT - 1

    @pl.when(wid == NW - 1)
    def _():
        tail_in_copy(1).start()

    in_copy(t_last, 0).wait()
    out_copy(c0_of(t_last - 2), 0).wait()
    transpose_block(0)
    out_copy(c0_of(t_last), 0).start()

    @pl.when(wid == NW - 1)
    def _():
        tail_in_copy(1).wait()
        out_copy(c0_of(t_last - 1), 1).wait()
        transpose_block(1)
        out_copy(TAIL_C0, 1).start()
        out_copy(TAIL_C0, 1).wait()

    @pl.when(wid != NW - 1)
    def _():
        out_copy(c0_of(t_last - 1), 1).wait()

    out_copy(c0_of(t_last), 0).wait()


@functools.partial(
    pl.kernel,
    mesh=_mesh,
    out_type=jax.ShapeDtypeStruct((B_TOT, ROW), jnp.float32),
    scratch_types=[pltpu.VMEM((CHUNK,), jnp.int32)] * NBUF
    + [pltpu.VMEM((CHUNK, EMBED), jnp.float32)] * NBUF
    + [pltpu.SemaphoreType.DMA] * (3 * NBUF),
    compiler_params=pltpu.CompilerParams(use_tc_tiling_on_sc=False),
)
def _embed_gather(table_hbm, idx_hbm, out_hbm, *scratch):
    idx_v = scratch[0:NBUF]
    rows_v = scratch[NBUF : 2 * NBUF]
    sems = scratch[2 * NBUF :]
    i_sem = sems[0:NBUF]
    g_sem = sems[NBUF : 2 * NBUF]
    s_sem = sems[2 * NBUF : 3 * NBUF]

    wid = lax.axis_index("s") * 2 + lax.axis_index("c")
    base = wid * B_PER_W

    def idx_copy(chunk, b):
        return pltpu.make_async_copy(
            idx_hbm.at[pl.ds(base + chunk * CHUNK, CHUNK)], idx_v[b], i_sem[b]
        )

    def gather_copy(b):
        return pltpu.make_async_copy(table_hbm.at[idx_v[b]], rows_v[b], g_sem[b])

    def store_copy(chunk, b):
        return pltpu.make_async_copy(
            rows_v[b],
            out_hbm.at[pl.ds(base + chunk * CHUNK, CHUNK), pl.ds(0, EMBED)],
            s_sem[b],
        )

    for b in range(NBUF):
        idx_copy(b, b).start()
    for b in range(NBUF):
        idx_copy(b, b).wait()
        gather_copy(b).start()

    def body(r, carry):
        for b in range(NBUF):
            g = r * NBUF + b
            gather_copy(b).wait()
            store_copy(g, b).start()
            idx_copy(g + NBUF, b).start()
            store_copy(g, b).wait()
            idx_copy(g + NBUF, b).wait()
            gather_copy(b).start()
        return carry

    lax.fori_loop(0, N_ROUNDS - 1, body, 0)

    last = (N_ROUNDS - 1) * NBUF
    for b in range(NBUF):
        gather_copy(b).wait()
        store_copy(last + b, b).start()
    for b in range(NBUF):
        store_copy(last + b, b).wait()


def kernel(sentence, table):
    idx = sentence.reshape(-1).astype(jnp.int32)
    tt = jnp.transpose(table)
    tail = lax.slice(tt, (0, TAIL_C0), (EMBED, VOCAB_N))
    flat = _transpose_table(tt, tail)
    t64 = flat.reshape(VOCAB_N, EMBED)
    out = _embed_gather(t64, idx)
    return out[:, :EMBED].reshape(sentence.shape + (EMBED,))


# speedup vs baseline: 2.9929x; 1.0301x over previous
import functools

import jax
import jax.numpy as jnp
from jax import lax
from jax.experimental import pallas as pl
from jax.experimental.pallas import tpu as pltpu
from jax.experimental.pallas import tpu_sc as plsc

VOCAB_N = 1000000
EMBED = 64
ROW = 128
B_TOT = 4096 * 200
NW = 32
B_PER_W = B_TOT // NW
CHUNK = 512
N_CHUNKS = B_PER_W // CHUNK
NBUF = 2
N_ROUNDS = N_CHUNKS // NBUF

BCOLS = 256
NBLK = VOCAB_N // BCOLS
TAIL_C0 = VOCAB_N - 128

_mesh = plsc.VectorSubcoreMesh(core_axis_name="c", subcore_axis_name="s")


@functools.partial(
    pl.kernel,
    mesh=_mesh,
    out_type=jax.ShapeDtypeStruct((VOCAB_N * EMBED,), jnp.float32),
    scratch_types=[pltpu.VMEM((EMBED, BCOLS), jnp.float32)] * 3
    + [pltpu.VMEM((BCOLS * EMBED,), jnp.float32)] * 3
    + [pltpu.SemaphoreType.DMA] * 6,
    compiler_params=pltpu.CompilerParams(
        use_tc_tiling_on_sc=True, needs_layout_passes=False
    ),
)
def _transpose_table(tt_hbm, tail_hbm, out_hbm, s0, s1, s2, d0, d1, d2, *sems):
    svmem = (s0, s1, s2)
    dvmem = (d0, d1, d2)
    i_sem = sems[0:3]
    o_sem = sems[3:6]

    wid = lax.axis_index("s") * 2 + lax.axis_index("c")

    iota = lax.iota(jnp.int32, 16)
    jslots = [(iota + k) & 15 for k in range(16)]

    def c0_of(t):
        blk = jnp.minimum(wid + NW * t, NBLK - 1)
        return pl.multiple_of(blk * BCOLS, BCOLS)

    def in_copy(t, b):
        return pltpu.make_async_copy(
            tt_hbm.at[:, pl.ds(c0_of(t), BCOLS)], svmem[b], i_sem[b]
        )

    def tail_in_copy(b):
        return pltpu.make_async_copy(
            tail_hbm.at[:, :], svmem[b].at[:, pl.ds(0, 128)], i_sem[b]
        )

    def out_copy(c0, b):
        return pltpu.make_async_copy(
            dvmem[b], out_hbm.at[pl.ds(c0 * EMBED, BCOLS * EMBED)], o_sem[b]
        )

    def tail_out_copy(b):
        return pltpu.make_async_copy(
            dvmem[b].at[pl.ds(0, 128 * EMBED)],
            out_hbm.at[pl.ds(TAIL_C0 * EMBED, 128 * EMBED)],
            o_sem[b],
        )

    def transpose_block(b, nid):
        s = svmem[b]
        d = dvmem[b]
        nrow = nid // 16

        @plsc.parallel_loop(0, nrow * 4, step=1, unroll=4)
        def _(sb):
            I = (sb % nrow) * 16
            J = (sb // nrow) * 16
            iv = I + iota
            dbase = iv * EMBED + J
            for k in range(16):
                jv = J + jslots[k]
                v = plsc.load_gather(s, [jv, iv])
                plsc.store_scatter(d, [dbase + jslots[k]], v)

    NT = -(-NBLK // NW)
    NTRI = NT // 3

    in_copy(0, 0).start()
    in_copy(1, 1).start()

    def body(r, carry):
        for bb in range(3):
            t = r * 3 + bb
            in_copy(t, bb).wait()

            @pl.when(t + 2 < NT)
            def _():
                in_copy(t + 2, (bb + 2) % 3).start()

            @pl.when(t >= 3)
            def _():
                out_copy(c0_of(t - 3), bb).wait()

            transpose_block(bb, BCOLS)
            out_copy(c0_of(t), bb).start()
        return carry

    lax.fori_loop(0, NTRI, body, 0)

    @pl.when(wid == NW - 1)
    def _():
        tail_in_copy(0).start()

    out_copy(c0_of(NT - 3), 0).wait()

    @pl.when(wid == NW - 1)
    def _():
        tail_in_copy(0).wait()
        transpose_block(0, 128)
        tail_out_copy(0).start()
        tail_out_copy(0).wait()

    out_copy(c0_of(NT - 2), 1).wait()
    out_copy(c0_of(NT - 1), 2).wait()


@functools.partial(
    pl.kernel,
    mesh=_mesh,
    out_type=jax.ShapeDtypeStruct((B_TOT, ROW), jnp.float32),
    scratch_types=[pltpu.VMEM((CHUNK,), jnp.int32)] * NBUF
    + [pltpu.VMEM((CHUNK, EMBED), jnp.float32)] * NBUF
    + [pltpu.SemaphoreType.DMA] * (3 * NBUF),
    compiler_params=pltpu.CompilerParams(use_tc_tiling_on_sc=False),
)
def _embed_gather(table_hbm, idx_hbm, out_hbm, *scratch):
    idx_v = scratch[0:NBUF]
    rows_v = scratch[NBUF : 2 * NBUF]
    sems = scratch[2 * NBUF :]
    i_sem = sems[0:NBUF]
    g_sem = sems[NBUF : 2 * NBUF]
    s_sem = sems[2 * NBUF : 3 * NBUF]

    wid = lax.axis_index("s") * 2 + lax.axis_index("c")
    base = wid * B_PER_W

    def idx_copy(chunk, b):
        return pltpu.make_async_copy(
            idx_hbm.at[pl.ds(base + chunk * CHUNK, CHUNK)], idx_v[b], i_sem[b]
        )

    def gather_copy(b):
        return pltpu.make_async_copy(table_hbm.at[idx_v[b]], rows_v[b], g_sem[b])

    def store_copy(chunk, b):
        return pltpu.make_async_copy(
            rows_v[b],
            out_hbm.at[pl.ds(base + chunk * CHUNK, CHUNK), pl.ds(0, EMBED)],
            s_sem[b],
        )

    for b in range(NBUF):
        idx_copy(b, b).start()
    for b in range(NBUF):
        idx_copy(b, b).wait()
        gather_copy(b).start()

    def body(r, carry):
        for b in range(NBUF):
            g = r * NBUF + b
            gather_copy(b).wait()
            store_copy(g, b).start()
            idx_copy(g + NBUF, b).start()
            store_copy(g, b).wait()
            idx_copy(g + NBUF, b).wait()
            gather_copy(b).start()
        return carry

    lax.fori_loop(0, N_ROUNDS - 1, body, 0)

    last = (N_ROUNDS - 1) * NBUF
    for b in range(NBUF):
        gather_copy(b).wait()
        store_copy(last + b, b).start()
    for b in range(NBUF):
        store_copy(last + b, b).wait()


def kernel(sentence, table):
    idx = sentence.reshape(-1).astype(jnp.int32)
    tt = jnp.transpose(table)
    tail = lax.slice(tt, (0, TAIL_C0), (EMBED, VOCAB_N))
    flat = _transpose_table(tt, tail)
    t64 = flat.reshape(VOCAB_N, EMBED)
    out = _embed_gather(t64, idx)
    return out[:, :EMBED].reshape(sentence.shape + (EMBED,))


# speedup vs baseline: 3.2514x; 1.0864x over previous
import functools

import jax
import jax.numpy as jnp
from jax import lax
from jax.experimental import pallas as pl
from jax.experimental.pallas import tpu as pltpu
from jax.experimental.pallas import tpu_sc as plsc

VOCAB_N = 1000000
EMBED = 64
ROW = 128
B_TOT = 4096 * 200
NW = 32
B_PER_W = B_TOT // NW
CHUNK = 512
N_CHUNKS = B_PER_W // CHUNK
NBUF = 2
N_ROUNDS = N_CHUNKS // NBUF

BCOLS = 256
NBLK = VOCAB_N // BCOLS
TAIL_C0 = VOCAB_N - 128

_mesh = plsc.VectorSubcoreMesh(core_axis_name="c", subcore_axis_name="s")


@functools.partial(
    pl.kernel,
    mesh=_mesh,
    out_type=jax.ShapeDtypeStruct((VOCAB_N * EMBED,), jnp.float32),
    scratch_types=[pltpu.VMEM((EMBED, BCOLS), jnp.float32)] * 3
    + [pltpu.VMEM((BCOLS * EMBED,), jnp.float32)] * 3
    + [pltpu.SemaphoreType.DMA] * 6,
    compiler_params=pltpu.CompilerParams(
        use_tc_tiling_on_sc=True, needs_layout_passes=False
    ),
)
def _transpose_table(tt_hbm, tail_hbm, out_hbm, s0, s1, s2, d0, d1, d2, *sems):
    svmem = (s0, s1, s2)
    dvmem = (d0, d1, d2)
    i_sem = sems[0:3]
    o_sem = sems[3:6]

    wid = lax.axis_index("s") * 2 + lax.axis_index("c")

    iota = lax.iota(jnp.int32, 16)
    jslots = [(iota + k) & 15 for k in range(16)]

    def c0_of(t):
        blk = jnp.minimum(wid + NW * t, NBLK - 1)
        return pl.multiple_of(blk * BCOLS, BCOLS)

    def in_copy(t, b):
        return pltpu.make_async_copy(
            tt_hbm.at[:, pl.ds(c0_of(t), BCOLS)], svmem[b], i_sem[b]
        )

    def tail_in_copy(b):
        return pltpu.make_async_copy(
            tail_hbm.at[:, :], svmem[b].at[:, pl.ds(0, 128)], i_sem[b]
        )

    def out_copy(c0, b):
        return pltpu.make_async_copy(
            dvmem[b], out_hbm.at[pl.ds(c0 * EMBED, BCOLS * EMBED)], o_sem[b]
        )

    def tail_out_copy(b):
        return pltpu.make_async_copy(
            dvmem[b].at[pl.ds(0, 128 * EMBED)],
            out_hbm.at[pl.ds(TAIL_C0 * EMBED, 128 * EMBED)],
            o_sem[b],
        )

    def transpose_block(b, nid):
        s = svmem[b]
        d = dvmem[b]
        nrow = nid // 16

        @plsc.parallel_loop(0, nrow * 4, step=1, unroll=8)
        def _(sb):
            I = (sb % nrow) * 16
            J = (sb // nrow) * 16
            iv = I + iota
            dbase = iv * EMBED + J
            for k in range(16):
                jv = J + jslots[k]
                v = plsc.load_gather(s, [jv, iv])
                plsc.store_scatter(d, [dbase + jslots[k]], v)

    NT = -(-NBLK // NW)
    NTRI = NT // 3

    in_copy(0, 0).start()
    in_copy(1, 1).start()

    def body(r, carry):
        for bb in range(3):
            t = r * 3 + bb
            in_copy(t, bb).wait()

            @pl.when(t + 2 < NT)
            def _():
                in_copy(t + 2, (bb + 2) % 3).start()

            @pl.when(t >= 3)
            def _():
                out_copy(c0_of(t - 3), bb).wait()

            transpose_block(bb, BCOLS)
            out_copy(c0_of(t), bb).start()
        return carry

    lax.fori_loop(0, NTRI, body, 0)

    @pl.when(wid == NW - 1)
    def _():
        tail_in_copy(0).start()

    out_copy(c0_of(NT - 3), 0).wait()

    @pl.when(wid == NW - 1)
    def _():
        tail_in_copy(0).wait()
        transpose_block(0, 128)
        tail_out_copy(0).start()
        tail_out_copy(0).wait()

    out_copy(c0_of(NT - 2), 1).wait()
    out_copy(c0_of(NT - 1), 2).wait()


@functools.partial(
    pl.kernel,
    mesh=_mesh,
    out_type=jax.ShapeDtypeStruct((B_TOT, ROW), jnp.float32),
    scratch_types=[pltpu.VMEM((CHUNK,), jnp.int32)] * NBUF
    + [pltpu.VMEM((CHUNK, EMBED), jnp.float32)] * NBUF
    + [pltpu.SemaphoreType.DMA] * (3 * NBUF),
    compiler_params=pltpu.CompilerParams(use_tc_tiling_on_sc=False),
)
def _embed_gather(table_hbm, idx_hbm, out_hbm, *scratch):
    idx_v = scratch[0:NBUF]
    rows_v = scratch[NBUF : 2 * NBUF]
    sems = scratch[2 * NBUF :]
    i_sem = sems[0:NBUF]
    g_sem = sems[NBUF : 2 * NBUF]
    s_sem = sems[2 * NBUF : 3 * NBUF]

    wid = lax.axis_index("s") * 2 + lax.axis_index("c")
    base = wid * B_PER_W

    def idx_copy(chunk, b):
        return pltpu.make_async_copy(
            idx_hbm.at[pl.ds(base + chunk * CHUNK, CHUNK)], idx_v[b], i_sem[b]
        )

    def gather_copy(b):
        return pltpu.make_async_copy(table_hbm.at[idx_v[b]], rows_v[b], g_sem[b])

    def store_copy(chunk, b):
        return pltpu.make_async_copy(
            rows_v[b],
            out_hbm.at[pl.ds(base + chunk * CHUNK, CHUNK), pl.ds(0, EMBED)],
            s_sem[b],
        )

    for b in range(NBUF):
        idx_copy(b, b).start()
    for b in range(NBUF):
        idx_copy(b, b).wait()
        gather_copy(b).start()

    def body(r, carry):
        for b in range(NBUF):
            g = r * NBUF + b
            gather_copy(b).wait()
            store_copy(g, b).start()
            idx_copy(g + NBUF, b).start()
            store_copy(g, b).wait()
            idx_copy(g + NBUF, b).wait()
            gather_copy(b).start()
        return carry

    lax.fori_loop(0, N_ROUNDS - 1, body, 0)

    last = (N_ROUNDS - 1) * NBUF
    for b in range(NBUF):
        gather_copy(b).wait()
        store_copy(last + b, b).start()
    for b in range(NBUF):
        store_copy(last + b, b).wait()


def kernel(sentence, table):
    idx = sentence.reshape(-1).astype(jnp.int32)
    tt = jnp.transpose(table)
    tail = lax.slice(tt, (0, TAIL_C0), (EMBED, VOCAB_N))
    flat = _transpose_table(tt, tail)
    t64 = flat.reshape(VOCAB_N, EMBED)
    out = _embed_gather(t64, idx)
    return out[:, :EMBED].reshape(sentence.shape + (EMBED,))
